# baseline (speedup 1.0000x reference)
import functools

import jax
import jax.numpy as jnp
from jax import lax
from jax.experimental import pallas as pl
from jax.experimental.pallas import tpu as pltpu
from jax.experimental.pallas import tpu_sc as plsc

_NC = 2
_NS = 16
_NW = _NC * _NS

_G = 128
_Q = 4
_K = 5
_RM = 2048
_RW = 10000


def _tc_widen(E):
    v, d = E.shape
    assert v % _RW == 0

    def widen_body(e_ref, out_ref):
        a = e_ref[...]
        out_ref[...] = jnp.concatenate([a, a], axis=1)

    return pl.pallas_call(
        widen_body,
        grid=(v // _RW,),
        in_specs=[pl.BlockSpec((_RW, d), lambda i: (i, 0))],
        out_specs=pl.BlockSpec((_RW, 2 * d), lambda i: (i, 0)),
        out_shape=jax.ShapeDtypeStruct((v, 2 * d), jnp.float32),
    )(E)


def _sc_gather(ids_2d, E_pad):
    n_chunks, g = ids_2d.shape
    assert g == _G
    v, dp = E_pad.shape
    assert n_chunks % (_NW * _K) == 0
    cpw = n_chunks // _NW
    n_blk = cpw // _K

    mesh = plsc.VectorSubcoreMesh(
        core_axis_name="c", subcore_axis_name="s",
        num_cores=_NC, num_subcores=_NS)

    @functools.partial(
        pl.kernel,
        out_type=jax.ShapeDtypeStruct((n_chunks * _G, dp), jnp.float32),
        mesh=mesh,
        scratch_types=[
            pltpu.VMEM((cpw, _G), jnp.int32),
            pltpu.VMEM((_K * _G, dp), jnp.float32),
            pltpu.SemaphoreType.DMA,
        ],
        compiler_params=pltpu.CompilerParams(use_tc_tiling_on_sc=False),
    )
    def gather_kernel(ids_hbm, table_hbm, emb_hbm, idx_v, rows_v, sem):
        wid = lax.axis_index("s") * _NC + lax.axis_index("c")
        chunk_base = wid * cpw
        row_base = chunk_base * _G
        pltpu.sync_copy(ids_hbm.at[pl.ds(chunk_base, cpw)], idx_v)

        def body(blk, carry):
            descs = []
            for k in range(_K):
                descs.append(pltpu.async_copy(
                    table_hbm.at[idx_v.at[blk * _K + k]],
                    rows_v.at[pl.ds(k * _G, _G)], sem))
            for dsc in descs:
                dsc.wait()
            pltpu.sync_copy(
                rows_v,
                emb_hbm.at[pl.ds(row_base + blk * _K * _G, _K * _G)])
            return carry

        lax.fori_loop(0, n_blk, body, 0)

    return gather_kernel(ids_2d, E_pad)


def _tc_project_chunk(out_prev, emb, p_wt, chunk_idx, n_total):
    rows, dp = emb.shape
    d = dp // 2
    m = p_wt.shape[1]
    assert rows % _RM == 0
    spc = rows // _RM
    out_shape = jax.ShapeDtypeStruct((n_total, m), jnp.float32)

    emb_spec = pl.BlockSpec((_RM, dp), lambda i: (i, 0))
    w_spec = pl.BlockSpec((d, m), lambda i: (0, 0))
    out_spec = pl.BlockSpec(
        (_RM, m), lambda i, _c=chunk_idx, _spc=spc: (_c * _spc + i, 0))

    if out_prev is None:
        def mm_first(emb_ref, w_ref, out_ref):
            out_ref[...] = jnp.dot(emb_ref[:, :d], w_ref[...],
                                   preferred_element_type=jnp.float32)

        return pl.pallas_call(
            mm_first,
            grid=(spc,),
            in_specs=[emb_spec, w_spec],
            out_specs=out_spec,
            out_shape=out_shape,
        )(emb, p_wt)

    def mm_next(carrier_ref, emb_ref, w_ref, out_ref):
        del carrier_ref
        out_ref[...] = jnp.dot(emb_ref[:, :d], w_ref[...],
                               preferred_element_type=jnp.float32)

    return pl.pallas_call(
        mm_next,
        grid=(spc,),
        in_specs=[
            pl.BlockSpec(memory_space=pl.ANY),
            emb_spec, w_spec,
        ],
        out_specs=out_spec,
        out_shape=out_shape,
        input_output_aliases={0: 0},
    )(out_prev, emb, p_wt)


def kernel(token_ids, E, P_w):
    b, l = token_ids.shape
    v, d = E.shape
    m = P_w.shape[0]
    n = b * l
    assert n % (_Q * _G) == 0

    ids_2d = token_ids.reshape(n // _G, _G).astype(jnp.int32)
    p_wt = P_w.T
    E_pad = _tc_widen(E)
    rows_per_chunk = n // _Q // _G

    out = None
    for q in range(_Q):
        ids_q = lax.slice_in_dim(ids_2d, q * rows_per_chunk,
                                 (q + 1) * rows_per_chunk, axis=0)
        emb = _sc_gather(ids_q, E_pad)
        out = _tc_project_chunk(out, emb, p_wt, q, n)

    return out.reshape(b, l, m)


# speedup vs baseline: 1.0018x; 1.0018x over previous
import functools

import jax
import jax.numpy as jnp
from jax import lax
from jax.experimental import pallas as pl
from jax.experimental.pallas import tpu as pltpu
from jax.experimental.pallas import tpu_sc as plsc

_NC = 2
_NS = 16
_NW = _NC * _NS

_G = 128
_Q = 4
_K = 5
_RM = 2048
_RW = 10000


def _tc_widen(E):
    v, d = E.shape
    assert v % _RW == 0

    def widen_body(e_ref, out_ref):
        a = e_ref[...]
        out_ref[...] = jnp.concatenate([a, a], axis=1)

    return pl.pallas_call(
        widen_body,
        grid=(v // _RW,),
        in_specs=[pl.BlockSpec((_RW, d), lambda i: (i, 0))],
        out_specs=pl.BlockSpec((_RW, 2 * d), lambda i: (i, 0)),
        out_shape=jax.ShapeDtypeStruct((v, 2 * d), jnp.float32),
    )(E)


def _sc_gather(ids_3d, E_pad):
    n_chunks, one, g = ids_3d.shape
    assert g == _G and one == 1
    v, dp = E_pad.shape
    assert n_chunks % (_NW * _K) == 0
    cpw = n_chunks // _NW
    n_blk = cpw // _K

    mesh = plsc.VectorSubcoreMesh(
        core_axis_name="c", subcore_axis_name="s",
        num_cores=_NC, num_subcores=_NS)

    @functools.partial(
        pl.kernel,
        out_type=jax.ShapeDtypeStruct((n_chunks * _G, dp), jnp.float32),
        mesh=mesh,
        scratch_types=[
            pltpu.VMEM((cpw, 1, _G), jnp.int32),
            pltpu.VMEM((_K * _G, dp), jnp.float32),
            pltpu.SemaphoreType.DMA,
        ],
    )
    def gather_kernel(ids_hbm, table_hbm, emb_hbm, idx_v, rows_v, sem):
        wid = lax.axis_index("s") * _NC + lax.axis_index("c")
        chunk_base = wid * cpw
        row_base = chunk_base * _G
        pltpu.sync_copy(ids_hbm.at[pl.ds(chunk_base, cpw)], idx_v)

        def body(blk, carry):
            descs = []
            for k in range(_K):
                descs.append(pltpu.async_copy(
                    table_hbm.at[idx_v.at[blk * _K + k, 0]],
                    rows_v.at[pl.ds(k * _G, _G)], sem))
            for dsc in descs:
                dsc.wait()
            pltpu.sync_copy(
                rows_v,
                emb_hbm.at[pl.ds(row_base + blk * _K * _G, _K * _G)])
            return carry

        lax.fori_loop(0, n_blk, body, 0)

    return gather_kernel(ids_3d, E_pad)


def _tc_project_chunk(out_prev, emb, p_wt, chunk_idx, n_total):
    rows, dp = emb.shape
    d = dp // 2
    m = p_wt.shape[1]
    assert rows % _RM == 0
    spc = rows // _RM
    out_shape = jax.ShapeDtypeStruct((n_total, m), jnp.float32)

    emb_spec = pl.BlockSpec((_RM, dp), lambda i: (i, 0))
    w_spec = pl.BlockSpec((d, m), lambda i: (0, 0))
    out_spec = pl.BlockSpec(
        (_RM, m), lambda i, _c=chunk_idx, _spc=spc: (_c * _spc + i, 0))

    if out_prev is None:
        def mm_first(emb_ref, w_ref, out_ref):
            out_ref[...] = jnp.dot(emb_ref[:, :d], w_ref[...],
                                   preferred_element_type=jnp.float32)

        return pl.pallas_call(
            mm_first,
            grid=(spc,),
            in_specs=[emb_spec, w_spec],
            out_specs=out_spec,
            out_shape=out_shape,
        )(emb, p_wt)

    def mm_next(carrier_ref, emb_ref, w_ref, out_ref):
        del carrier_ref
        out_ref[...] = jnp.dot(emb_ref[:, :d], w_ref[...],
                               preferred_element_type=jnp.float32)

    return pl.pallas_call(
        mm_next,
        grid=(spc,),
        in_specs=[
            pl.BlockSpec(memory_space=pl.ANY),
            emb_spec, w_spec,
        ],
        out_specs=out_spec,
        out_shape=out_shape,
        input_output_aliases={0: 0},
    )(out_prev, emb, p_wt)


def kernel(token_ids, E, P_w):
    b, l = token_ids.shape
    v, d = E.shape
    m = P_w.shape[0]
    n = b * l
    assert n % (_Q * _G) == 0

    ids_3d = token_ids.reshape(n // _G, 1, _G).astype(jnp.int32)
    p_wt = P_w.T
    E_pad = _tc_widen(E)
    rows_per_chunk = n // _Q // _G

    out = None
    for q in range(_Q):
        ids_q = lax.slice_in_dim(ids_3d, q * rows_per_chunk,
                                 (q + 1) * rows_per_chunk, axis=0)
        emb = _sc_gather(ids_q, E_pad)
        out = _tc_project_chunk(out, emb, p_wt, q, n)

    return out.reshape(b, l, m)


# speedup vs baseline: 1.0879x; 1.0859x over previous
import functools

import jax
import jax.numpy as jnp
from jax import lax
from jax.experimental import pallas as pl
from jax.experimental.pallas import tpu as pltpu
from jax.experimental.pallas import tpu_sc as plsc

_NC = 2
_NS = 16
_NW = _NC * _NS

_G = 128
_Q = 4
_K = 5
_RM = 2048


def _sc_gather_packed(ids_2d, E_pad):
    n_chunks, g = ids_2d.shape
    assert g == _G
    v, dp = E_pad.shape
    d = dp // 2
    assert n_chunks % (_NW * _K) == 0
    cpw = n_chunks // _NW
    n_blk = cpw // _K
    half_rows = n_chunks * _G // 2

    mesh = plsc.VectorSubcoreMesh(
        core_axis_name="c", subcore_axis_name="s",
        num_cores=_NC, num_subcores=_NS)

    @functools.partial(
        pl.kernel,
        out_type=jax.ShapeDtypeStruct((half_rows, dp), jnp.float32),
        mesh=mesh,
        scratch_types=[
            pltpu.VMEM((cpw, _G), jnp.int32),
            pltpu.VMEM((_K * _G, dp), jnp.float32),
            pltpu.SemaphoreType.DMA,
        ],
        compiler_params=pltpu.CompilerParams(use_tc_tiling_on_sc=False),
    )
    def gather_kernel(ids_hbm, table_hbm, emb_hbm, idx_v, rows_v, sem):
        c = lax.axis_index("c")
        s = lax.axis_index("s")
        chunk_base = (c * _NS + s) * cpw
        row_base = s * cpw * _G
        pltpu.sync_copy(ids_hbm.at[pl.ds(chunk_base, cpw)], idx_v)

        def body(blk, carry):
            descs = []
            for k in range(_K):
                descs.append(pltpu.async_copy(
                    table_hbm.at[idx_v.at[blk * _K + k]],
                    rows_v.at[pl.ds(k * _G, _G)], sem))
            for dsc in descs:
                dsc.wait()
            r0 = row_base + blk * _K * _G
            src = rows_v.at[:, pl.ds(0, d)]

            @pl.when(c == 0)
            def _():
                pltpu.sync_copy(
                    src, emb_hbm.at[pl.ds(r0, _K * _G), pl.ds(0, d)])

            @pl.when(c == 1)
            def _():
                pltpu.sync_copy(
                    src, emb_hbm.at[pl.ds(r0, _K * _G), pl.ds(d, d)])

            return carry

        lax.fori_loop(0, n_blk, body, 0)

    return gather_kernel(ids_2d, E_pad)


def _tc_project_chunk(out_prev, emb2, p_wt, chunk_idx, num_chunks):
    nh, dp = emb2.shape
    d = dp // 2
    m = p_wt.shape[1]
    assert nh % _RM == 0
    out_shape = jax.ShapeDtypeStruct((2 * num_chunks, nh, m), jnp.float32)

    emb_spec = pl.BlockSpec((_RM, dp), lambda i: (i, 0))
    w_spec = pl.BlockSpec((d, m), lambda i: (0, 0))
    out_spec = pl.BlockSpec(
        (2, _RM, m), lambda i, _c=chunk_idx: (_c, i, 0))

    if out_prev is None:
        def mm_first(emb_ref, w_ref, out_ref):
            a = emb_ref[...]
            w = w_ref[...]
            out_ref[0] = jnp.dot(a[:, :d], w,
                                 preferred_element_type=jnp.float32)
            out_ref[1] = jnp.dot(a[:, d:], w,
                                 preferred_element_type=jnp.float32)

        return pl.pallas_call(
            mm_first,
            grid=(nh // _RM,),
            in_specs=[emb_spec, w_spec],
            out_specs=out_spec,
            out_shape=out_shape,
        )(emb2, p_wt)

    def mm_next(carrier_ref, emb_ref, w_ref, out_ref):
        del carrier_ref
        a = emb_ref[...]
        w = w_ref[...]
        out_ref[0] = jnp.dot(a[:, :d], w, preferred_element_type=jnp.float32)
        out_ref[1] = jnp.dot(a[:, d:], w, preferred_element_type=jnp.float32)

    return pl.pallas_call(
        mm_next,
        grid=(nh // _RM,),
        in_specs=[
            pl.BlockSpec(memory_space=pl.ANY),
            emb_spec, w_spec,
        ],
        out_specs=out_spec,
        out_shape=out_shape,
        input_output_aliases={0: 0},
    )(out_prev, emb2, p_wt)


def kernel(token_ids, E, P_w):
    b, l = token_ids.shape
    v, d = E.shape
    m = P_w.shape[0]
    n = b * l
    assert n % (_Q * _G) == 0

    ids_2d = token_ids.reshape(n // _G, _G).astype(jnp.int32)
    p_wt = P_w.T
    E_pad = jnp.concatenate([E, E], axis=1)
    rows_per_chunk = n // _Q // _G

    out = None
    for q in range(_Q):
        ids_q = lax.slice_in_dim(ids_2d, q * rows_per_chunk,
                                 (q + 1) * rows_per_chunk, axis=0)
        emb2 = _sc_gather_packed(ids_q, E_pad)
        out = _tc_project_chunk(out, emb2, p_wt, q, _Q)

    return out.reshape(b, l, m)


# speedup vs baseline: 1.1992x; 1.1023x over previous
import functools

import jax
import jax.numpy as jnp
from jax import lax
from jax.experimental import pallas as pl
from jax.experimental.pallas import tpu as pltpu
from jax.experimental.pallas import tpu_sc as plsc

_NC = 2
_NS = 16
_NW = _NC * _NS

_G = 128
_Q = 2
_K = 5
_RM = 2048


def _sc_gather_packed(ids_2d, E):
    n_chunks, g = ids_2d.shape
    assert g == _G
    v, d = E.shape
    assert n_chunks % (_NW * _K) == 0
    cpw = n_chunks // _NW
    n_blk = cpw // _K
    half_rows = n_chunks * _G // 2

    mesh = plsc.VectorSubcoreMesh(
        core_axis_name="c", subcore_axis_name="s",
        num_cores=_NC, num_subcores=_NS)

    @functools.partial(
        pl.kernel,
        out_type=jax.ShapeDtypeStruct((half_rows, 2 * d), jnp.float32),
        mesh=mesh,
        scratch_types=[
            pltpu.VMEM((cpw, _G), jnp.int32),
            pltpu.VMEM((_K * _G, d), jnp.float32),
            pltpu.SemaphoreType.DMA,
        ],
        compiler_params=pltpu.CompilerParams(use_tc_tiling_on_sc=False),
    )
    def gather_kernel(ids_hbm, table_hbm, emb_hbm, idx_v, rows_v, sem):
        c = lax.axis_index("c")
        s = lax.axis_index("s")
        chunk_base = (c * _NS + s) * cpw
        row_base = s * cpw * _G
        pltpu.sync_copy(ids_hbm.at[pl.ds(chunk_base, cpw)], idx_v)

        def body(blk, carry):
            descs = []
            for k in range(_K):
                descs.append(pltpu.async_copy(
                    table_hbm.at[idx_v.at[blk * _K + k]],
                    rows_v.at[pl.ds(k * _G, _G)], sem))
            for dsc in descs:
                dsc.wait()
            r0 = row_base + blk * _K * _G

            @pl.when(c == 0)
            def _():
                pltpu.sync_copy(
                    rows_v, emb_hbm.at[pl.ds(r0, _K * _G), pl.ds(0, d)])

            @pl.when(c == 1)
            def _():
                pltpu.sync_copy(
                    rows_v, emb_hbm.at[pl.ds(r0, _K * _G), pl.ds(d, d)])

            return carry

        lax.fori_loop(0, n_blk, body, 0)

    return gather_kernel(ids_2d, E)


def _tc_project_chunk(out_prev, emb2, p_wt, chunk_idx, num_chunks):
    nh, dp = emb2.shape
    d = dp // 2
    m = p_wt.shape[1]
    assert nh % _RM == 0
    out_shape = jax.ShapeDtypeStruct((2 * num_chunks, nh, m), jnp.float32)

    emb_spec = pl.BlockSpec((_RM, dp), lambda i: (i, 0))
    w_spec = pl.BlockSpec((d, m), lambda i: (0, 0))
    out_spec = pl.BlockSpec(
        (2, _RM, m), lambda i, _c=chunk_idx: (_c, i, 0))

    if out_prev is None:
        def mm_first(emb_ref, w_ref, out_ref):
            a = emb_ref[...]
            w = w_ref[...]
            out_ref[0] = jnp.dot(a[:, :d], w,
                                 preferred_element_type=jnp.float32)
            out_ref[1] = jnp.dot(a[:, d:], w,
                                 preferred_element_type=jnp.float32)

        return pl.pallas_call(
            mm_first,
            grid=(nh // _RM,),
            in_specs=[emb_spec, w_spec],
            out_specs=out_spec,
            out_shape=out_shape,
        )(emb2, p_wt)

    def mm_next(carrier_ref, emb_ref, w_ref, out_ref):
        del carrier_ref
        a = emb_ref[...]
        w = w_ref[...]
        out_ref[0] = jnp.dot(a[:, :d], w, preferred_element_type=jnp.float32)
        out_ref[1] = jnp.dot(a[:, d:], w, preferred_element_type=jnp.float32)

    return pl.pallas_call(
        mm_next,
        grid=(nh // _RM,),
        in_specs=[
            pl.BlockSpec(memory_space=pl.ANY),
            emb_spec, w_spec,
        ],
        out_specs=out_spec,
        out_shape=out_shape,
        input_output_aliases={0: 0},
    )(out_prev, emb2, p_wt)


def kernel(token_ids, E, P_w):
    b, l = token_ids.shape
    v, d = E.shape
    m = P_w.shape[0]
    n = b * l
    assert n % (_Q * _G) == 0

    ids_2d = token_ids.reshape(n // _G, _G).astype(jnp.int32)
    p_wt = P_w.T
    rows_per_chunk = n // _Q // _G

    out = None
    for q in range(_Q):
        ids_q = lax.slice_in_dim(ids_2d, q * rows_per_chunk,
                                 (q + 1) * rows_per_chunk, axis=0)
        emb2 = _sc_gather_packed(ids_q, E)
        out = _tc_project_chunk(out, emb2, p_wt, q, _Q)

    return out.reshape(b, l, m)
